# baseline (speedup 1.0000x reference)
import functools

import jax
import jax.numpy as jnp
from jax import lax
from jax.experimental import pallas as pl
from jax.experimental.pallas import tpu as pltpu
from jax.experimental.pallas import tpu_sc as plsc

N_NODES = 10000
N_PAD = 10240
N_EDGES = 320000
NC = 2
NS = 16
KS = 128
PAD_NODE = N_PAD - 1
E_PAD = 327680
NCH1 = E_PAD // NS // KS
NCHS = NCH1 // NC
ROWS_PER_TILE = N_PAD // NS
ROW_BLOCK = 2000

_mesh = plsc.VectorSubcoreMesh(core_axis_name="c", subcore_axis_name="s")


def _deg_body(edges_hbm, zeros_hbm, out_hbm, deg_sh, didx, ones_v):
    c = lax.axis_index("c")
    s = lax.axis_index("s")
    w = 640 * s
    pltpu.sync_copy(zeros_hbm.at[pl.ds(w, 640)], deg_sh.at[pl.ds(w, 640)])
    pltpu.sync_copy(edges_hbm.at[1].at[s], didx)
    for i in range(KS // 16):
        ones_v[pl.ds(i * 16, 16)] = jnp.full((16,), 1.0, jnp.float32)
    plsc.subcore_barrier()

    def body(j, carry):
        pltpu.sync_copy(ones_v, deg_sh.at[didx.at[j]], add=True)
        return carry

    lax.fori_loop(0, NCH1, body, 0)
    plsc.subcore_barrier()

    @pl.when(c == 0)
    def _():
        pltpu.sync_copy(deg_sh.at[pl.ds(w, 640)],
                        out_hbm.at[pl.ds(w, 640)])


_deg_kernel = pl.kernel(
    _deg_body,
    out_type=jax.ShapeDtypeStruct((N_PAD,), jnp.float32),
    mesh=_mesh,
    compiler_params=pltpu.CompilerParams(use_tc_tiling_on_sc=False),
    scratch_types=[
        pltpu.VMEM_SHARED((N_PAD,), jnp.float32),
        pltpu.VMEM((NCH1, KS), jnp.int32),
        pltpu.VMEM((KS,), jnp.float32),
    ],
)


NBUF = 5


def _segsum_ring(h_hbm, edges_hbm, zeros_hbm, out_hbm,
                 acc, sidx, didx, rows0, rows1, rows2, rows3, rows4,
                 g0, g1, g2, g3, g4, s0, s1, s2, s3, s4,
                 *, nch, feat_split):
    rows = (rows0, rows1, rows2, rows3, rows4)
    gsem = (g0, g1, g2, g3, g4)
    ssem = (s0, s1, s2, s3, s4)
    c = lax.axis_index("c")
    s = lax.axis_index("s")
    r0 = ROWS_PER_TILE * s
    if feat_split:
        table = h_hbm.at[c]
        my_src = edges_hbm.at[0].at[s]
        my_dst = edges_hbm.at[1].at[s]
    else:
        table = h_hbm
        my_src = edges_hbm.at[0].at[s].at[pl.ds(c * NCHS, NCHS)]
        my_dst = edges_hbm.at[1].at[s].at[pl.ds(c * NCHS, NCHS)]
    pltpu.sync_copy(zeros_hbm, acc.at[pl.ds(r0, ROWS_PER_TILE)])
    pltpu.sync_copy(my_src, sidx)
    pltpu.sync_copy(my_dst, didx)
    plsc.subcore_barrier()
    pltpu.async_copy(table.at[sidx.at[0]], rows0, g0)
    pltpu.async_copy(table.at[sidx.at[1]], rows1, g1)

    def body(t, carry):
        for q in range(NBUF):
            j = NBUF * t + q
            pltpu.make_async_copy(table.at[sidx.at[j]], rows[q],
                                  gsem[q]).wait()
            pltpu.async_copy(rows[q], acc.at[didx.at[j]], ssem[q], add=True)
            q2 = (q + 2) % NBUF
            if q < NBUF - 2:
                @pl.when(t > 0)
                def _():
                    pltpu.make_async_copy(
                        rows[q2], acc.at[didx.at[0]], ssem[q2]).wait()
                pltpu.async_copy(table.at[sidx.at[j + 2]], rows[q2], gsem[q2])
            else:
                @pl.when(j + 2 < nch)
                def _():
                    pltpu.make_async_copy(
                        rows[q2], acc.at[didx.at[0]], ssem[q2]).wait()
                    pltpu.async_copy(table.at[sidx.at[j + 2]], rows[q2],
                                     gsem[q2])
        return carry

    lax.fori_loop(0, nch // NBUF, body, 0)
    for q in range(NBUF):
        pltpu.make_async_copy(rows[q], acc.at[didx.at[0]], ssem[q]).wait()
    plsc.subcore_barrier()
    pltpu.sync_copy(acc.at[pl.ds(r0, ROWS_PER_TILE)],
                    out_hbm.at[c].at[pl.ds(r0, ROWS_PER_TILE)])


def _make_segsum(F, *, feat_split):
    nch = NCH1 if feat_split else NCHS
    body = functools.partial(_segsum_ring, nch=nch, feat_split=feat_split)
    return pl.kernel(
        body,
        out_type=jax.ShapeDtypeStruct((NC, N_PAD, F), jnp.float32),
        mesh=_mesh,
        compiler_params=pltpu.CompilerParams(use_tc_tiling_on_sc=False),
        scratch_types=(
            [pltpu.VMEM_SHARED((N_PAD, F), jnp.float32),
             pltpu.VMEM((nch, KS), jnp.int32),
             pltpu.VMEM((nch, KS), jnp.int32)]
            + [pltpu.VMEM((KS, F), jnp.float32) for _ in range(NBUF)]
            + [pltpu.SemaphoreType.DMA for _ in range(2 * NBUF)]
        ),
    )


_segsum64 = _make_segsum(64, feat_split=True)
_segsum40 = _make_segsum(40, feat_split=False)


def _stage1_body(deg_ref, x_ref, w_ref, h_ref, dinv_ref):
    deg = deg_ref[...]
    dinv = jnp.where(deg > 0, lax.rsqrt(deg), 0.0)
    h = jnp.dot(x_ref[...], w_ref[...], preferred_element_type=jnp.float32)
    h = h * dinv
    h_ref[0] = h[:, :64]
    h_ref[1] = h[:, 64:]
    dinv_ref[...] = dinv


def _stage2_body(a0_ref, a1_ref, dinv_ref, b_ref, w_ref, out_ref):
    dinv = dinv_ref[...]
    a = jnp.concatenate([a0_ref[0], a1_ref[0]], axis=1)
    s = a * dinv + b_ref[...]
    h = jnp.maximum(s, 0.0)
    out_ref[...] = jnp.dot(h, w_ref[...],
                           preferred_element_type=jnp.float32) * dinv


def _stage3_body(a0_ref, a1_ref, dinv_ref, b_ref, out_ref):
    out_ref[...] = ((a0_ref[0] + a1_ref[0]) * dinv_ref[...] + b_ref[...])


def _row_spec(f):
    return pl.BlockSpec((ROW_BLOCK, f), lambda i: (i, 0))


def _half_spec(cidx, f):
    return pl.BlockSpec((1, ROW_BLOCK, f), lambda i, _c=cidx: (_c, i, 0))


def _full_spec(r, f):
    return pl.BlockSpec((r, f), lambda i: (0, 0))


_GRID = (N_NODES // ROW_BLOCK,)

_stage1 = pl.pallas_call(
    _stage1_body,
    grid=_GRID,
    in_specs=[_row_spec(1), _row_spec(128), _full_spec(128, 128)],
    out_specs=[pl.BlockSpec((2, ROW_BLOCK, 64), lambda i: (0, i, 0)),
               _row_spec(1)],
    out_shape=[jax.ShapeDtypeStruct((2, N_PAD, 64), jnp.float32),
               jax.ShapeDtypeStruct((N_NODES, 1), jnp.float32)],
)

_stage2 = pl.pallas_call(
    _stage2_body,
    grid=_GRID,
    in_specs=[_half_spec(0, 64), _half_spec(1, 64), _row_spec(1),
              _full_spec(1, 128), _full_spec(128, 40)],
    out_specs=_row_spec(40),
    out_shape=jax.ShapeDtypeStruct((N_PAD, 40), jnp.float32),
)

_stage3 = pl.pallas_call(
    _stage3_body,
    grid=_GRID,
    in_specs=[_half_spec(0, 40), _half_spec(1, 40), _row_spec(1),
              _full_spec(1, 40)],
    out_specs=_row_spec(40),
    out_shape=jax.ShapeDtypeStruct((N_NODES, 40), jnp.float32),
)


def kernel(x, edge_index, W0, b0, W1, b1):
    pad_row = N_NODES + jnp.arange(E_PAD - N_EDGES, dtype=jnp.int32) % (
        N_PAD - N_NODES)
    pad = jnp.broadcast_to(pad_row, (2, E_PAD - N_EDGES))
    e4 = jnp.concatenate([edge_index, pad], axis=1).reshape(2, NS, NCH1, KS)
    zeros_n = jnp.zeros((N_PAD,), jnp.float32)
    zeros64 = jnp.zeros((ROWS_PER_TILE, 64), jnp.float32)
    zeros40 = jnp.zeros((ROWS_PER_TILE, 40), jnp.float32)

    deg = _deg_kernel(e4, zeros_n).reshape(N_PAD, 1)
    h1p, dinv = _stage1(deg, x, W0.T)
    acc1 = _segsum64(h1p, e4, zeros64)
    h2p = _stage2(acc1, acc1, dinv, b0.reshape(1, 128), W1.T)
    acc2 = _segsum40(h2p, e4, zeros40)
    return _stage3(acc2, acc2, dinv, b1.reshape(1, 40))


# speedup vs baseline: 1.0009x; 1.0009x over previous
import functools

import jax
import jax.numpy as jnp
from jax import lax
from jax.experimental import pallas as pl
from jax.experimental.pallas import tpu as pltpu
from jax.experimental.pallas import tpu_sc as plsc

N_NODES = 10000
N_PAD = 10240
N_EDGES = 320000
NC = 2
NS = 16
KS = 128
E_PAD = 327680
NCH1 = E_PAD // NS // KS
NCHS = NCH1 // NC
ROWS_PER_TILE = N_PAD // NS
ROW_BLOCK = 2000

_mesh = plsc.VectorSubcoreMesh(core_axis_name="c", subcore_axis_name="s")


def _deg_body(edges_hbm, zeros_hbm, out_hbm, deg_sh, didx, ones_v):
    c = lax.axis_index("c")
    s = lax.axis_index("s")
    w = 640 * s
    pltpu.sync_copy(zeros_hbm.at[pl.ds(w, 640)], deg_sh.at[pl.ds(w, 640)])
    pltpu.sync_copy(edges_hbm.at[1].at[s], didx)
    for i in range(KS // 16):
        ones_v[pl.ds(i * 16, 16)] = jnp.full((16,), 1.0, jnp.float32)
    plsc.subcore_barrier()

    def body(j, carry):
        pltpu.sync_copy(ones_v, deg_sh.at[didx.at[j]], add=True)
        return carry

    lax.fori_loop(0, NCH1, body, 0)
    plsc.subcore_barrier()

    @pl.when(c == 0)
    def _():
        pltpu.sync_copy(deg_sh.at[pl.ds(w, 640)],
                        out_hbm.at[pl.ds(w, 640)])


_deg_kernel = pl.kernel(
    _deg_body,
    out_type=jax.ShapeDtypeStruct((N_PAD,), jnp.float32),
    mesh=_mesh,
    compiler_params=pltpu.CompilerParams(use_tc_tiling_on_sc=False),
    scratch_types=[
        pltpu.VMEM_SHARED((N_PAD,), jnp.float32),
        pltpu.VMEM((NCH1, KS), jnp.int32),
        pltpu.VMEM((KS,), jnp.float32),
    ],
)


NBUF = 5


def _segsum_ring(h_hbm, edges_hbm, zeros_hbm, out_hbm,
                 acc, sidx, didx, rows0, rows1, rows2, rows3, rows4,
                 g0, g1, g2, g3, g4, s0, s1, s2, s3, s4,
                 *, nch, feat_split):
    rows = (rows0, rows1, rows2, rows3, rows4)
    gsem = (g0, g1, g2, g3, g4)
    ssem = (s0, s1, s2, s3, s4)
    c = lax.axis_index("c")
    s = lax.axis_index("s")
    r0 = ROWS_PER_TILE * s
    if feat_split:
        table = h_hbm.at[c]
        my_src = edges_hbm.at[0].at[s]
        my_dst = edges_hbm.at[1].at[s]
    else:
        table = h_hbm
        my_src = edges_hbm.at[0].at[s].at[pl.ds(c * NCHS, NCHS)]
        my_dst = edges_hbm.at[1].at[s].at[pl.ds(c * NCHS, NCHS)]
    pltpu.sync_copy(zeros_hbm, acc.at[pl.ds(r0, ROWS_PER_TILE)])
    pltpu.sync_copy(my_src, sidx)
    pltpu.sync_copy(my_dst, didx)
    plsc.subcore_barrier()
    pltpu.async_copy(table.at[sidx.at[0]], rows0, g0)
    pltpu.async_copy(table.at[sidx.at[1]], rows1, g1)

    def body(t, carry):
        for q in range(NBUF):
            j = NBUF * t + q
            pltpu.make_async_copy(table.at[sidx.at[j]], rows[q],
                                  gsem[q]).wait()
            pltpu.async_copy(rows[q], acc.at[didx.at[j]], ssem[q], add=True)
            q2 = (q + 2) % NBUF
            if q < NBUF - 2:
                @pl.when(t > 0)
                def _():
                    pltpu.make_async_copy(
                        rows[q2], acc.at[didx.at[0]], ssem[q2]).wait()
                pltpu.async_copy(table.at[sidx.at[j + 2]], rows[q2], gsem[q2])
            else:
                @pl.when(j + 2 < nch)
                def _():
                    pltpu.make_async_copy(
                        rows[q2], acc.at[didx.at[0]], ssem[q2]).wait()
                    pltpu.async_copy(table.at[sidx.at[j + 2]], rows[q2],
                                     gsem[q2])
        return carry

    lax.fori_loop(0, nch // NBUF, body, 0)
    for q in range(NBUF):
        pltpu.make_async_copy(rows[q], acc.at[didx.at[0]], ssem[q]).wait()
    plsc.subcore_barrier()
    pltpu.sync_copy(acc.at[pl.ds(r0, ROWS_PER_TILE)],
                    out_hbm.at[c].at[pl.ds(r0, ROWS_PER_TILE)])


def _make_segsum(F, *, feat_split):
    nch = NCH1 if feat_split else NCHS
    body = functools.partial(_segsum_ring, nch=nch, feat_split=feat_split)
    return pl.kernel(
        body,
        out_type=jax.ShapeDtypeStruct((NC, N_PAD, F), jnp.float32),
        mesh=_mesh,
        compiler_params=pltpu.CompilerParams(use_tc_tiling_on_sc=False),
        scratch_types=(
            [pltpu.VMEM_SHARED((N_PAD, F), jnp.float32),
             pltpu.VMEM((nch, KS), jnp.int32),
             pltpu.VMEM((nch, KS), jnp.int32)]
            + [pltpu.VMEM((KS, F), jnp.float32) for _ in range(NBUF)]
            + [pltpu.SemaphoreType.DMA for _ in range(2 * NBUF)]
        ),
    )


_segsum64 = _make_segsum(64, feat_split=True)
_segsum40 = _make_segsum(40, feat_split=False)


def _stage1_body(deg_ref, x_ref, w_ref, h_ref, dinv_ref):
    deg = deg_ref[...]
    dinv = jnp.where(deg > 0, lax.rsqrt(deg), 0.0)
    h = jnp.dot(x_ref[...], w_ref[...], preferred_element_type=jnp.float32)
    h = h * dinv
    h_ref[0] = h[:, :64]
    h_ref[1] = h[:, 64:]
    dinv_ref[...] = dinv


def _stage2_body(a0_ref, a1_ref, dinv_ref, b_ref, w_ref, out_ref):
    dinv = dinv_ref[...]
    a = jnp.concatenate([a0_ref[0], a1_ref[0]], axis=1)
    s = a * dinv + b_ref[...]
    h = jnp.maximum(s, 0.0)
    out_ref[...] = jnp.dot(h, w_ref[...],
                           preferred_element_type=jnp.float32) * dinv


def _stage3_body(a0_ref, a1_ref, dinv_ref, b_ref, out_ref):
    out_ref[...] = ((a0_ref[0] + a1_ref[0]) * dinv_ref[...] + b_ref[...])


def _row_spec(f):
    return pl.BlockSpec((ROW_BLOCK, f), lambda i: (i, 0))


def _half_spec(cidx, f):
    return pl.BlockSpec((1, ROW_BLOCK, f), lambda i, _c=cidx: (_c, i, 0))


def _full_spec(r, f):
    return pl.BlockSpec((r, f), lambda i: (0, 0))


_GRID = (N_NODES // ROW_BLOCK,)

_stage1 = pl.pallas_call(
    _stage1_body,
    grid=_GRID,
    in_specs=[_row_spec(1), _row_spec(128), _full_spec(128, 128)],
    out_specs=[pl.BlockSpec((2, ROW_BLOCK, 64), lambda i: (0, i, 0)),
               _row_spec(1)],
    out_shape=[jax.ShapeDtypeStruct((2, N_PAD, 64), jnp.float32),
               jax.ShapeDtypeStruct((N_NODES, 1), jnp.float32)],
)

_stage2 = pl.pallas_call(
    _stage2_body,
    grid=_GRID,
    in_specs=[_half_spec(0, 64), _half_spec(1, 64), _row_spec(1),
              _full_spec(1, 128), _full_spec(128, 40)],
    out_specs=_row_spec(40),
    out_shape=jax.ShapeDtypeStruct((N_PAD, 40), jnp.float32),
)

_stage3 = pl.pallas_call(
    _stage3_body,
    grid=_GRID,
    in_specs=[_half_spec(0, 40), _half_spec(1, 40), _row_spec(1),
              _full_spec(1, 40)],
    out_specs=_row_spec(40),
    out_shape=jax.ShapeDtypeStruct((N_NODES, 40), jnp.float32),
)


def kernel(x, edge_index, W0, b0, W1, b1):
    pad_row = N_NODES + jnp.arange(E_PAD - N_EDGES, dtype=jnp.int32) % (
        N_PAD - N_NODES)
    pad = jnp.broadcast_to(pad_row, (2, E_PAD - N_EDGES))
    e4 = jnp.concatenate([edge_index, pad], axis=1).reshape(2, NS, NCH1, KS)
    zeros_n = jnp.zeros((N_PAD,), jnp.float32)
    zeros64 = jnp.zeros((ROWS_PER_TILE, 64), jnp.float32)
    zeros40 = jnp.zeros((ROWS_PER_TILE, 40), jnp.float32)

    deg = _deg_kernel(e4, zeros_n).reshape(N_PAD, 1)
    h1p, dinv = _stage1(deg, x, W0.T)
    acc1 = _segsum64(h1p, e4, zeros64)
    h2p = _stage2(acc1, acc1, dinv, b0.reshape(1, 128), W1.T)
    acc2 = _segsum40(h2p, e4, zeros40)
    return _stage3(acc2, acc2, dinv, b1.reshape(1, 40))
